# baseline (speedup 1.0000x reference)
import jax
import jax.numpy as jnp
from jax import lax
from jax.experimental import pallas as pl
from jax.experimental.pallas import tpu as pltpu
from jax.experimental.pallas import tpu_sc as plsc

_B = 16384
_V = 1000
_NC = 2
_NS = 16
_L = 16
_NW = _NC * _NS
_ROWS_PER_W = _B // _NW
_R = 32
_CH = _ROWS_PER_W // _R


def _onehot_body(x_hbm, out_hbm, x_v, buf0, buf1, sem0, sem1):
    wid = lax.axis_index("s") * _NC + lax.axis_index("c")
    base = wid * _ROWS_PER_W

    pltpu.sync_copy(x_hbm.at[pl.ds(base * 1, _ROWS_PER_W)], x_v)

    bufs = (buf0, buf1)
    sems = (sem0, sem1)

    zrow = jnp.zeros((_L,), jnp.float32)

    @pl.loop(0, _R)
    def _zf(r):
        for buf in bufs:
            for c in range(0, _V - _L + 1, _L):
                buf[r, pl.ds(c, _L)] = zrow
            buf[r, pl.ds(_V - _L, _L)] = zrow

    ones = jnp.ones((_L,), jnp.float32)
    zeros = jnp.zeros((_L,), jnp.float32)
    lane = lax.iota(jnp.int32, _L)

    def _scatter(buf, ch, val):
        for j in range(_R // _L):
            rloc = lane + (j * _L)
            xv = x_v[pl.ds(ch * _R + j * _L, _L)]
            plsc.store_scatter(buf, [rloc, xv], val)

    handles = [None, None]
    for ch in range(_CH):
        b = ch % 2
        if handles[b] is not None:
            handles[b].wait()
            _scatter(bufs[b], ch - 2, zeros)
        _scatter(bufs[b], ch, ones)
        handles[b] = pltpu.async_copy(
            bufs[b], out_hbm.at[pl.ds(base + ch * _R, _R)], sems[b]
        )
    handles[_CH % 2].wait()
    handles[(_CH + 1) % 2].wait()


def kernel(x):
    mesh = plsc.VectorSubcoreMesh(core_axis_name="c", subcore_axis_name="s")
    run = pl.kernel(
        _onehot_body,
        out_type=jax.ShapeDtypeStruct((_B, _V), jnp.float32),
        mesh=mesh,
        compiler_params=pltpu.CompilerParams(
            needs_layout_passes=False, use_tc_tiling_on_sc=True
        ),
        scratch_types=[
            pltpu.VMEM((_ROWS_PER_W,), jnp.int32),
            pltpu.VMEM((_R, _V), jnp.float32),
            pltpu.VMEM((_R, _V), jnp.float32),
            pltpu.SemaphoreType.DMA,
            pltpu.SemaphoreType.DMA,
        ],
    )
    return run(x.astype(jnp.int32))


# speedup vs baseline: 1.9645x; 1.9645x over previous
import jax
import jax.numpy as jnp
from jax import lax
from jax.experimental import pallas as pl
from jax.experimental.pallas import tpu as pltpu
from jax.experimental.pallas import tpu_sc as plsc

_B = 16384
_V = 1000
_NC = 2
_NS = 16
_L = 16
_NW = _NC * _NS
_CB = _B // _NW
_RC = 96
_CHUNKS = [(c0, _RC) for c0 in range(0, _V - _RC + 1, _RC)] + [(960, 40)]


def _onehot_body(x_hbm, out_hbm, x_v, buf0, buf1, sem0, sem1):
    wid = lax.axis_index("s") * _NC + lax.axis_index("c")
    b0 = wid * _CB

    pltpu.sync_copy(x_hbm.at[pl.ds(b0, _CB)], x_v)

    bufs = (buf0, buf1)
    sems = (sem0, sem1)

    zrow = jnp.zeros((_L,), jnp.float32)

    @pl.loop(0, _RC)
    def _zf(r):
        for buf in bufs:
            for c in range(0, _CB, _L):
                buf[r, pl.ds(c, _L)] = zrow

    ones = jnp.ones((_L,), jnp.float32)
    zeros = jnp.zeros((_L,), jnp.float32)
    lane = lax.iota(jnp.int32, _L)

    def _scatter(buf, c0, nc, val):
        for j in range(_CB // _L):
            xv = x_v[pl.ds(j * _L, _L)]
            colv = lane + (j * _L)
            mask = (xv >= c0) & (xv < c0 + nc)
            plsc.store_scatter(buf, [xv - c0, colv], val, mask=mask)

    handles = [None, None]
    prev = [None, None]
    for ch, (c0, nc) in enumerate(_CHUNKS):
        b = ch % 2
        if handles[b] is not None:
            handles[b].wait()
            pc0, pnc = prev[b]
            _scatter(bufs[b], pc0, pnc, zeros)
        _scatter(bufs[b], c0, nc, ones)
        src = bufs[b] if nc == _RC else bufs[b].at[pl.ds(0, nc)]
        handles[b] = pltpu.async_copy(
            src, out_hbm.at[pl.ds(c0, nc), pl.ds(b0, _CB)], sems[b]
        )
        prev[b] = (c0, nc)
    handles[len(_CHUNKS) % 2].wait()
    handles[(len(_CHUNKS) + 1) % 2].wait()


def kernel(x):
    mesh = plsc.VectorSubcoreMesh(core_axis_name="c", subcore_axis_name="s")
    run = pl.kernel(
        _onehot_body,
        out_type=jax.ShapeDtypeStruct((_V, _B), jnp.float32),
        mesh=mesh,
        compiler_params=pltpu.CompilerParams(
            needs_layout_passes=False, use_tc_tiling_on_sc=True
        ),
        scratch_types=[
            pltpu.VMEM((_CB,), jnp.int32),
            pltpu.VMEM((_RC, _CB), jnp.float32),
            pltpu.VMEM((_RC, _CB), jnp.float32),
            pltpu.SemaphoreType.DMA,
            pltpu.SemaphoreType.DMA,
        ],
    )
    return run(x.astype(jnp.int32)).T


# speedup vs baseline: 2.2013x; 1.1205x over previous
import jax
import jax.numpy as jnp
from jax import lax
from jax.experimental import pallas as pl
from jax.experimental.pallas import tpu as pltpu
from jax.experimental.pallas import tpu_sc as plsc

_B = 16384
_V = 1000
_NC = 2
_NS = 16
_L = 16
_NW = _NC * _NS
_CB = _B // _NW
_RC = 48
_NBUF = 4
_CHUNKS = [(c0, _RC) for c0 in range(0, _V - _RC + 1, _RC)] + [(960, 40)]


def _onehot_body(x_hbm, out_hbm, x_v, *bufs_sems):
    bufs = bufs_sems[:_NBUF]
    sems = bufs_sems[_NBUF:]

    wid = lax.axis_index("s") * _NC + lax.axis_index("c")
    b0 = wid * _CB

    pltpu.sync_copy(x_hbm.at[pl.ds(b0, _CB)], x_v)

    zrow = jnp.zeros((_L,), jnp.float32)

    @pl.loop(0, _RC)
    def _zf(r):
        for buf in bufs:
            for c in range(0, _CB, _L):
                buf[r, pl.ds(c, _L)] = zrow

    ones = jnp.ones((_L,), jnp.float32)
    zeros = jnp.zeros((_L,), jnp.float32)
    lane = lax.iota(jnp.int32, _L)

    def _scatter(buf, c0, nc, val):
        @pl.loop(0, _CB // _L)
        def _sc_j(j):
            xv = x_v[pl.ds(j * _L, _L)]
            colv = lane + j * _L
            mask = (xv >= c0) & (xv < c0 + nc)
            plsc.store_scatter(buf, [xv - c0, colv], val, mask=mask)

    handles = [None] * _NBUF
    prev = [None] * _NBUF
    for ch, (c0, nc) in enumerate(_CHUNKS):
        b = ch % _NBUF
        if handles[b] is not None:
            handles[b].wait()
            pc0, pnc = prev[b]
            _scatter(bufs[b], pc0, pnc, zeros)
        _scatter(bufs[b], c0, nc, ones)
        src = bufs[b] if nc == _RC else bufs[b].at[pl.ds(0, nc)]
        handles[b] = pltpu.async_copy(
            src, out_hbm.at[pl.ds(c0, nc), pl.ds(b0, _CB)], sems[b]
        )
        prev[b] = (c0, nc)
    for b in range(_NBUF):
        if handles[b] is not None:
            handles[b].wait()


def kernel(x):
    mesh = plsc.VectorSubcoreMesh(core_axis_name="c", subcore_axis_name="s")
    run = pl.kernel(
        _onehot_body,
        out_type=jax.ShapeDtypeStruct((_V, _B), jnp.float32),
        mesh=mesh,
        compiler_params=pltpu.CompilerParams(
            needs_layout_passes=False, use_tc_tiling_on_sc=True
        ),
        scratch_types=(
            [pltpu.VMEM((_CB,), jnp.int32)]
            + [pltpu.VMEM((_RC, _CB), jnp.float32)] * _NBUF
            + [pltpu.SemaphoreType.DMA] * _NBUF
        ),
    )
    return run(x.astype(jnp.int32)).T


# speedup vs baseline: 2.2317x; 1.0138x over previous
import jax
import jax.numpy as jnp
from jax import lax
from jax.experimental import pallas as pl
from jax.experimental.pallas import tpu as pltpu
from jax.experimental.pallas import tpu_sc as plsc

_B = 16384
_V = 1000
_NC = 2
_NS = 16
_L = 16
_NW = _NC * _NS
_NG = 16
_CB = _B // _NG
_RC = 56
_NBUF = 2
_CHUNKS0 = [(c0, _RC) for c0 in range(0, 504 - _RC + 1, _RC)]
_CHUNKS1 = [(c0, _RC) for c0 in range(504, _V - _RC + 1, _RC)] + [(952, 48)]


def _onehot_body(x_hbm, out_hbm, x_v, *bufs_sems):
    bufs = bufs_sems[:_NBUF]
    sems = bufs_sems[_NBUF:]

    wid = lax.axis_index("s") * _NC + lax.axis_index("c")
    grp = wid % _NG
    half = wid // _NG
    b0 = grp * _CB

    pltpu.sync_copy(x_hbm.at[pl.ds(b0, _CB)], x_v)

    zrow = jnp.zeros((_L,), jnp.float32)

    @pl.loop(0, _RC)
    def _zf(r):
        for buf in bufs:
            for c in range(0, _CB, _L):
                buf[r, pl.ds(c, _L)] = zrow

    ones = jnp.ones((_L,), jnp.float32)
    zeros = jnp.zeros((_L,), jnp.float32)
    lane = lax.iota(jnp.int32, _L)

    def _scatter(buf, c0, nc, val):
        @pl.loop(0, _CB // _L)
        def _sc_j(j):
            xv = x_v[pl.ds(j * _L, _L)]
            colv = lane + j * _L
            mask = (xv >= c0) & (xv < c0 + nc)
            plsc.store_scatter(buf, [xv - c0, colv], val, mask=mask)

    cbase = half * 504
    handles = [None] * _NBUF
    prev = [None] * _NBUF
    for ch in range(8):
        b = ch % _NBUF
        c0 = cbase + ch * _RC
        if handles[b] is not None:
            handles[b].wait()
            _scatter(bufs[b], prev[b], _RC, zeros)
        _scatter(bufs[b], c0, _RC, ones)
        handles[b] = pltpu.async_copy(
            bufs[b], out_hbm.at[pl.ds(c0, _RC), pl.ds(b0, _CB)], sems[b]
        )
        prev[b] = c0
    handles[0].wait()
    _scatter(bufs[0], prev[0], _RC, zeros)
    c0 = cbase + 8 * _RC
    nc = jnp.where(half == 0, _RC, 48)
    _scatter(bufs[0], c0, nc, ones)

    @pl.when(half == 0)
    def _last0():
        pltpu.sync_copy(bufs[0], out_hbm.at[pl.ds(c0, _RC), pl.ds(b0, _CB)])

    @pl.when(half == 1)
    def _last1():
        pltpu.sync_copy(
            bufs[0].at[pl.ds(0, 48)],
            out_hbm.at[pl.ds(c0, 48), pl.ds(b0, _CB)],
        )

    handles[1].wait()


def kernel(x):
    mesh = plsc.VectorSubcoreMesh(core_axis_name="c", subcore_axis_name="s")
    run = pl.kernel(
        _onehot_body,
        out_type=jax.ShapeDtypeStruct((_V, _B), jnp.float32),
        mesh=mesh,
        compiler_params=pltpu.CompilerParams(
            needs_layout_passes=False, use_tc_tiling_on_sc=True
        ),
        scratch_types=(
            [pltpu.VMEM((_CB,), jnp.int32)]
            + [pltpu.VMEM((_RC, _CB), jnp.float32)] * _NBUF
            + [pltpu.SemaphoreType.DMA] * _NBUF
        ),
    )
    return run(x.astype(jnp.int32)).T


# speedup vs baseline: 2.3292x; 1.0437x over previous
import jax
import jax.numpy as jnp
from jax import lax
from jax.experimental import pallas as pl
from jax.experimental.pallas import tpu as pltpu
from jax.experimental.pallas import tpu_sc as plsc

_B = 16384
_V = 1000
_NC = 2
_NS = 16
_L = 16
_NW = _NC * _NS
_NG = 16
_CB = _B // _NG
_RC = 56
_NBUF = 2
_CHUNKS0 = [(c0, _RC) for c0 in range(0, 504 - _RC + 1, _RC)]
_CHUNKS1 = [(c0, _RC) for c0 in range(504, _V - _RC + 1, _RC)] + [(952, 48)]


def _onehot_body(x_hbm, out_hbm, x_v, *bufs_sems):
    bufs = bufs_sems[:_NBUF]
    sems = bufs_sems[_NBUF:]

    wid = lax.axis_index("s") * _NC + lax.axis_index("c")
    grp = wid % _NG
    half = wid // _NG
    b0 = grp * _CB

    pltpu.sync_copy(x_hbm.at[pl.ds(b0, _CB)], x_v)

    zrow = jnp.zeros((_L,), jnp.float32)

    def _zf(buf):
        @pl.loop(0, _RC)
        def _zf_r(r):
            for c in range(0, _CB, _L):
                buf[r, pl.ds(c, _L)] = zrow

    ones = jnp.ones((_L,), jnp.float32)
    zeros = jnp.zeros((_L,), jnp.float32)
    lane = lax.iota(jnp.int32, _L)

    def _scatter(buf, c0, nc, val):
        @pl.loop(0, _CB // _L)
        def _sc_j(j):
            xv = x_v[pl.ds(j * _L, _L)]
            colv = lane + j * _L
            mask = (xv >= c0) & (xv < c0 + nc)
            plsc.store_scatter(buf, [xv - c0, colv], val, mask=mask)

    cbase = half * 504
    handles = [None] * _NBUF
    prev = [None] * _NBUF
    for ch in range(8):
        b = ch % _NBUF
        c0 = cbase + ch * _RC
        if handles[b] is not None:
            handles[b].wait()
            _scatter(bufs[b], prev[b], _RC, zeros)
        elif ch < _NBUF:
            _zf(bufs[b])
        _scatter(bufs[b], c0, _RC, ones)
        handles[b] = pltpu.async_copy(
            bufs[b], out_hbm.at[pl.ds(c0, _RC), pl.ds(b0, _CB)], sems[b]
        )
        prev[b] = c0
    handles[0].wait()
    _scatter(bufs[0], prev[0], _RC, zeros)
    c0 = cbase + 8 * _RC
    nc = jnp.where(half == 0, _RC, 48)
    _scatter(bufs[0], c0, nc, ones)

    @pl.when(half == 0)
    def _last0():
        pltpu.sync_copy(bufs[0], out_hbm.at[pl.ds(c0, _RC), pl.ds(b0, _CB)])

    @pl.when(half == 1)
    def _last1():
        pltpu.sync_copy(
            bufs[0].at[pl.ds(0, 48)],
            out_hbm.at[pl.ds(c0, 48), pl.ds(b0, _CB)],
        )

    handles[1].wait()


def kernel(x):
    mesh = plsc.VectorSubcoreMesh(core_axis_name="c", subcore_axis_name="s")
    run = pl.kernel(
        _onehot_body,
        out_type=jax.ShapeDtypeStruct((_V, _B), jnp.float32),
        mesh=mesh,
        compiler_params=pltpu.CompilerParams(
            needs_layout_passes=False, use_tc_tiling_on_sc=True
        ),
        scratch_types=(
            [pltpu.VMEM((_CB,), jnp.int32)]
            + [pltpu.VMEM((_RC, _CB), jnp.float32)] * _NBUF
            + [pltpu.SemaphoreType.DMA] * _NBUF
        ),
    )
    return run(x.astype(jnp.int32)).T
